# baseline (speedup 1.0000x reference)
import functools

import jax
import jax.numpy as jnp
from jax import lax
from jax.experimental import pallas as pl
from jax.experimental.pallas import tpu as pltpu
from jax.experimental.pallas import tpu_sc as plsc

N = 10000
E = 320000
D = 128
NP = 10240
NC = 2
NS = 16
NW = NC * NS
K = 64
CH = 157
EPT = CH * K
EP = EPT * NW
NB = 4
NBI = 8
PF = 2
PFI = 3
DEG_CH = 4
DEG_CHUNK = EPT // DEG_CH
ROWS_PT = NP // NS
BR = 512
F32 = jnp.float32


def _sc_mesh():
    return plsc.VectorSubcoreMesh(core_axis_name="c", subcore_axis_name="s")


def _deg_partials(col_p, ew_p):
    @functools.partial(
        pl.kernel,
        out_type=jax.ShapeDtypeStruct((NW, NP), F32),
        mesh=_sc_mesh(),
        compiler_params=pltpu.CompilerParams(needs_layout_passes=False),
        scratch_types=[
            pltpu.VMEM((NP,), F32),
            pltpu.VMEM((DEG_CHUNK,), jnp.int32),
            pltpu.VMEM((DEG_CHUNK,), F32),
        ],
    )
    def k(col_hbm, ew_hbm, deg_hbm, dbuf, colb, ewb):
        wid = lax.axis_index("c") * NS + lax.axis_index("s")
        z16 = jnp.zeros((16,), F32)

        def zero_body(i, _):
            dbuf[pl.ds(i * 16, 16)] = z16
            return 0

        lax.fori_loop(0, NP // 16, zero_body, 0, unroll=8)

        base0 = wid * EPT
        for ch in range(DEG_CH):
            src = pl.ds(base0 + ch * DEG_CHUNK, DEG_CHUNK)
            pltpu.sync_copy(col_hbm.at[src], colb)
            pltpu.sync_copy(ew_hbm.at[src], ewb)

            def grp_body(g, _):
                c16 = colb[pl.ds(g * 16, 16)]
                w16 = ewb[pl.ds(g * 16, 16)]
                plsc.addupdate_scatter(dbuf, [c16], w16)
                return 0

            lax.fori_loop(0, DEG_CHUNK // 16, grp_body, 0, unroll=4)
        pltpu.sync_copy(dbuf, deg_hbm.at[wid])

    return k(col_p, ew_p)


def _aggregate(hp, row_p, col3_p, ew_p):
    @functools.partial(
        pl.kernel,
        out_type=jax.ShapeDtypeStruct((NC, NP, D), F32),
        mesh=_sc_mesh(),
        compiler_params=pltpu.CompilerParams(needs_layout_passes=False),
        scratch_types=[
            pltpu.MemorySpace.VMEM_SHARED((NP, D), F32),
            pltpu.VMEM((NBI, K), jnp.int32),
            pltpu.VMEM((NBI, 1, K), jnp.int32),
            pltpu.VMEM((NBI, K), F32),
            pltpu.VMEM((NB, K, D), F32),
            pltpu.SemaphoreType.DMA((NBI,)),
            pltpu.SemaphoreType.DMA((NB,)),
            pltpu.SemaphoreType.DMA((NB,)),
        ],
    )
    def k(hp_hbm, row_hbm, col_hbm, ew_hbm, out_hbm, acc, rowb, colb, ewb,
          gbuf, isem, gsem, ssem):
        cid = lax.axis_index("c")
        sid = lax.axis_index("s")
        wid = cid * NS + sid
        z16 = jnp.zeros((16,), F32)

        def zrow(i, _):
            for r in range(8):
                gbuf[0, i, pl.ds(r * 16, 16)] = z16
            return 0

        lax.fori_loop(0, K, zrow, 0)
        zbase = sid * ROWS_PT
        for b in range(ROWS_PT // K):
            pltpu.sync_copy(gbuf.at[0], acc.at[pl.ds(zbase + b * K, K)])
        rem_rows = ROWS_PT - (ROWS_PT // K) * K
        if rem_rows:
            pltpu.sync_copy(gbuf.at[0, pl.ds(0, rem_rows)],
                            acc.at[pl.ds(zbase + (ROWS_PT // K) * K,
                                         rem_rows)])
        plsc.subcore_barrier()

        def fire_idx(t, bi):
            base = pl.ds(wid * EPT + t * K, K)
            pltpu.async_copy(row_hbm.at[base], rowb.at[bi], isem.at[bi])
            pltpu.async_copy(col_hbm.at[wid, t], colb.at[bi], isem.at[bi])
            pltpu.async_copy(ew_hbm.at[base], ewb.at[bi], isem.at[bi])

        def wait_idx(t, bi):
            base = pl.ds(wid * EPT + t * K, K)
            pltpu.make_async_copy(row_hbm.at[base], rowb.at[bi],
                                  isem.at[bi]).wait()
            pltpu.make_async_copy(col_hbm.at[wid, t], colb.at[bi],
                                  isem.at[bi]).wait()
            pltpu.make_async_copy(ew_hbm.at[base], ewb.at[bi],
                                  isem.at[bi]).wait()

        def fire_gather(t, b, bi):
            pltpu.async_copy(hp_hbm.at[rowb.at[bi]], gbuf.at[b], gsem.at[b])

        def wait_gather(t, b, bi):
            pltpu.make_async_copy(hp_hbm.at[rowb.at[bi]], gbuf.at[b],
                                  gsem.at[b]).wait()

        def fire_scatter(t, b, bi):
            pltpu.async_copy(gbuf.at[b], acc.at[colb.at[bi, 0]], ssem.at[b],
                             add=True)

        def wait_scatter(t, b, bi):
            pltpu.make_async_copy(gbuf.at[b], acc.at[colb.at[bi, 0]],
                                  ssem.at[b]).wait()

        for t in range(PFI):
            fire_idx(t, t % NBI)
        for t in range(PF):
            wait_idx(t, t % NBI)
            fire_gather(t, t % NB, t % NBI)

        def chunk_body(t, _):
            b = lax.rem(t, NB)
            bi = lax.rem(t, NBI)

            @pl.when(t + PFI < CH)
            def _prefetch_idx():
                fire_idx(t + PFI, lax.rem(t + PFI, NBI))

            @pl.when(t + PF < CH)
            def _prefetch_gather():
                tn = t + PF

                @pl.when(t >= PF)
                def _drain():
                    wait_scatter(t - PF, lax.rem(tn, NB),
                                 lax.rem(t - PF, NBI))

                wait_idx(tn, lax.rem(tn, NBI))
                fire_gather(tn, lax.rem(tn, NB), lax.rem(tn, NBI))

            wait_gather(t, b, bi)

            for g in range(K // 16):
                w16 = ewb[bi, pl.ds(g * 16, 16)]
                for j in range(16):
                    wv = jnp.take_along_axis(
                        w16, jnp.full((16,), j, jnp.int32), axis=0)
                    e = g * 16 + j
                    for r in range(D // 16):
                        gbuf[b, e, pl.ds(r * 16, 16)] = (
                            gbuf[b, e, pl.ds(r * 16, 16)] * wv)
            fire_scatter(t, b, bi)
            return 0

        lax.fori_loop(0, CH, chunk_body, 0)
        for t in range(CH - NB, CH):
            wait_scatter(t, t % NB, t % NBI)
        plsc.subcore_barrier()
        rows = pl.ds(sid * ROWS_PT, ROWS_PT)
        pltpu.sync_copy(acc.at[rows], out_hbm.at[cid, rows])

    return k(hp, row_p, col3_p, ew_p)


def _dis_of(dg_block):
    d = jnp.sum(dg_block, axis=0) + 1.0
    return jnp.where(d > 0, lax.rsqrt(d), 0.0)


def _mm_scale(xp, W, degp):
    def body(x_ref, w_ref, dg_ref, o_ref):
        dis = _dis_of(dg_ref[...])
        h = jnp.dot(x_ref[...], w_ref[...], preferred_element_type=F32)
        o_ref[...] = h * dis[:, None]

    return pl.pallas_call(
        body,
        grid=(NP // BR,),
        in_specs=[
            pl.BlockSpec((BR, D), lambda i: (i, 0)),
            pl.BlockSpec((D, D), lambda i: (0, 0)),
            pl.BlockSpec((NW, BR), lambda i: (0, i)),
        ],
        out_specs=pl.BlockSpec((BR, D), lambda i: (i, 0)),
        out_shape=jax.ShapeDtypeStruct((NP, D), F32),
    )(xp, W, degp)


def _agg_relu_mm_scale(parts, hp, degp, W):
    def body(p_ref, hp_ref, dg_ref, w_ref, o_ref):
        dis = _dis_of(dg_ref[...])
        s = jnp.sum(p_ref[...], axis=0) + hp_ref[...]
        z = jnp.maximum(s * dis[:, None], 0.0)
        h = jnp.dot(z, w_ref[...], preferred_element_type=F32)
        o_ref[...] = h * dis[:, None]

    return pl.pallas_call(
        body,
        grid=(NP // BR,),
        in_specs=[
            pl.BlockSpec((NC, BR, D), lambda i: (0, i, 0)),
            pl.BlockSpec((BR, D), lambda i: (i, 0)),
            pl.BlockSpec((NW, BR), lambda i: (0, i)),
            pl.BlockSpec((D, D), lambda i: (0, 0)),
        ],
        out_specs=pl.BlockSpec((BR, D), lambda i: (i, 0)),
        out_shape=jax.ShapeDtypeStruct((NP, D), F32),
    )(parts, hp, degp, W)


def _agg_log_softmax(parts, hp, degp):
    def body(p_ref, hp_ref, dg_ref, o_ref):
        dis = _dis_of(dg_ref[...])
        s = jnp.sum(p_ref[...], axis=0) + hp_ref[...]
        agg = s * dis[:, None]
        m = jnp.max(agg, axis=1, keepdims=True)
        sh = agg - m
        lse = jnp.log(jnp.sum(jnp.exp(sh), axis=1, keepdims=True))
        o_ref[...] = sh - lse

    return pl.pallas_call(
        body,
        grid=(NP // BR,),
        in_specs=[
            pl.BlockSpec((NC, BR, D), lambda i: (0, i, 0)),
            pl.BlockSpec((BR, D), lambda i: (i, 0)),
            pl.BlockSpec((NW, BR), lambda i: (0, i)),
        ],
        out_specs=pl.BlockSpec((BR, D), lambda i: (i, 0)),
        out_shape=jax.ShapeDtypeStruct((NP, D), F32),
    )(parts, hp, degp)


def kernel(x, edge_index, edge_weight, W1, W2):
    row = edge_index[0].astype(jnp.int32)
    col = edge_index[1].astype(jnp.int32)
    pad_e = EP - E
    row_p = jnp.concatenate([row, jnp.zeros((pad_e,), jnp.int32)])
    col_p = jnp.concatenate([col, jnp.zeros((pad_e,), jnp.int32)])
    ew_p = jnp.concatenate([edge_weight.astype(F32), jnp.zeros((pad_e,), F32)])
    xp = jnp.concatenate([x.astype(F32), jnp.zeros((NP - N, D), F32)], axis=0)
    col3_p = col_p.reshape(NW, CH, 1, K)

    degp = _deg_partials(col_p, ew_p)
    hp1 = _mm_scale(xp, W1, degp)
    parts1 = _aggregate(hp1, row_p, col3_p, ew_p)
    hp2 = _agg_relu_mm_scale(parts1, hp1, degp, W2)
    parts2 = _aggregate(hp2, row_p, col3_p, ew_p)
    outp = _agg_log_softmax(parts2, hp2, degp)
    return outp[:N]


# speedup vs baseline: 1.1016x; 1.1016x over previous
import functools

import jax
import jax.numpy as jnp
from jax import lax
from jax.experimental import pallas as pl
from jax.experimental.pallas import tpu as pltpu
from jax.experimental.pallas import tpu_sc as plsc

N = 10000
E = 320000
D = 128
NP = 10240
NC = 2
NS = 16
NW = NC * NS
K = 48
CH = 209
EPT = CH * K
EP = EPT * NW
NB = 3
NBI = 6
PF = 2
PFI = 3
DEG_CH = 3
DEG_CHUNK = EPT // DEG_CH
ROWS_PT = NP // NS
BR = 512
F32 = jnp.float32


def _sc_mesh():
    return plsc.VectorSubcoreMesh(core_axis_name="c", subcore_axis_name="s")


def _deg_partials(col_p, ew_p):
    @functools.partial(
        pl.kernel,
        out_type=jax.ShapeDtypeStruct((NW, NP), F32),
        mesh=_sc_mesh(),
        compiler_params=pltpu.CompilerParams(needs_layout_passes=False),
        scratch_types=[
            pltpu.VMEM((NP,), F32),
            pltpu.VMEM((DEG_CHUNK,), jnp.int32),
            pltpu.VMEM((DEG_CHUNK,), F32),
        ],
    )
    def k(col_hbm, ew_hbm, deg_hbm, dbuf, colb, ewb):
        wid = lax.axis_index("c") * NS + lax.axis_index("s")
        z16 = jnp.zeros((16,), F32)

        def zero_body(i, _):
            dbuf[pl.ds(i * 16, 16)] = z16
            return 0

        lax.fori_loop(0, NP // 16, zero_body, 0, unroll=8)

        base0 = wid * EPT
        for ch in range(DEG_CH):
            src = pl.ds(base0 + ch * DEG_CHUNK, DEG_CHUNK)
            pltpu.sync_copy(col_hbm.at[src], colb)
            pltpu.sync_copy(ew_hbm.at[src], ewb)

            def grp_body(g, _):
                c16 = colb[pl.ds(g * 16, 16)]
                w16 = ewb[pl.ds(g * 16, 16)]
                plsc.addupdate_scatter(dbuf, [c16], w16)
                return 0

            lax.fori_loop(0, DEG_CHUNK // 16, grp_body, 0, unroll=4)
        pltpu.sync_copy(dbuf, deg_hbm.at[wid])

    return k(col_p, ew_p)


def _aggregate(hp, row_p, col3_p, ew_p):
    @functools.partial(
        pl.kernel,
        out_type=jax.ShapeDtypeStruct((NC, NP, D), F32),
        mesh=_sc_mesh(),
        compiler_params=pltpu.CompilerParams(needs_layout_passes=False),
        scratch_types=[
            pltpu.MemorySpace.VMEM_SHARED((NP, D), F32),
            pltpu.VMEM((NBI, K), jnp.int32),
            pltpu.VMEM((NBI, 1, K), jnp.int32),
            pltpu.VMEM((NBI, K), F32),
            pltpu.VMEM((NB, K, D), F32),
            pltpu.VMEM((NB, K, D), F32),
            pltpu.SemaphoreType.DMA((NBI,)),
            pltpu.SemaphoreType.DMA((NB,)),
            pltpu.SemaphoreType.DMA((NB,)),
        ],
    )
    def k(hp_hbm, row_hbm, col_hbm, ew_hbm, out_hbm, acc, rowb, colb, ewb,
          gbuf, sbuf, isem, gsem, ssem):
        cid = lax.axis_index("c")
        sid = lax.axis_index("s")
        wid = cid * NS + sid
        z16 = jnp.zeros((16,), F32)

        def zrow(i, _):
            for r in range(8):
                gbuf[0, i, pl.ds(r * 16, 16)] = z16
            return 0

        lax.fori_loop(0, K, zrow, 0)
        zbase = sid * ROWS_PT
        for b in range(ROWS_PT // K):
            pltpu.sync_copy(gbuf.at[0], acc.at[pl.ds(zbase + b * K, K)])
        rem_rows = ROWS_PT - (ROWS_PT // K) * K
        if rem_rows:
            pltpu.sync_copy(gbuf.at[0, pl.ds(0, rem_rows)],
                            acc.at[pl.ds(zbase + (ROWS_PT // K) * K,
                                         rem_rows)])
        plsc.subcore_barrier()

        def fire_idx(t, bi):
            base = pl.ds(wid * EPT + t * K, K)
            pltpu.async_copy(row_hbm.at[base], rowb.at[bi], isem.at[bi])
            pltpu.async_copy(col_hbm.at[wid, t], colb.at[bi], isem.at[bi])
            pltpu.async_copy(ew_hbm.at[base], ewb.at[bi], isem.at[bi])

        def wait_idx(t, bi):
            base = pl.ds(wid * EPT + t * K, K)
            pltpu.make_async_copy(row_hbm.at[base], rowb.at[bi],
                                  isem.at[bi]).wait()
            pltpu.make_async_copy(col_hbm.at[wid, t], colb.at[bi],
                                  isem.at[bi]).wait()
            pltpu.make_async_copy(ew_hbm.at[base], ewb.at[bi],
                                  isem.at[bi]).wait()

        def fire_gather(t, b, bi):
            pltpu.async_copy(hp_hbm.at[rowb.at[bi]], gbuf.at[b], gsem.at[b])

        def wait_gather(t, b, bi):
            pltpu.make_async_copy(hp_hbm.at[rowb.at[bi]], gbuf.at[b],
                                  gsem.at[b]).wait()

        def fire_scatter(t, b, bi):
            pltpu.async_copy(sbuf.at[b], acc.at[colb.at[bi, 0]], ssem.at[b],
                             add=True)

        def wait_scatter(t, b, bi):
            pltpu.make_async_copy(sbuf.at[b], acc.at[colb.at[bi, 0]],
                                  ssem.at[b]).wait()

        for t in range(PFI):
            fire_idx(t, t % NBI)
        for t in range(PF):
            wait_idx(t, t % NBI)
            fire_gather(t, t % NB, t % NBI)

        def chunk_body(t, _):
            b = lax.rem(t, NB)
            bi = lax.rem(t, NBI)

            @pl.when(t + PFI < CH)
            def _prefetch_idx():
                fire_idx(t + PFI, lax.rem(t + PFI, NBI))

            @pl.when(t + PF < CH)
            def _prefetch_gather():
                tn = t + PF
                wait_idx(tn, lax.rem(tn, NBI))
                fire_gather(tn, lax.rem(tn, NB), lax.rem(tn, NBI))

            wait_gather(t, b, bi)

            @pl.when(t >= NB)
            def _drain():
                wait_scatter(t - NB, b, lax.rem(t - NB, NBI))

            for g in range(K // 16):
                w16 = ewb[bi, pl.ds(g * 16, 16)]
                for j in range(16):
                    wv = jnp.take_along_axis(
                        w16, jnp.full((16,), j, jnp.int32), axis=0)
                    e = g * 16 + j
                    for r in range(D // 16):
                        sbuf[b, e, pl.ds(r * 16, 16)] = (
                            gbuf[b, e, pl.ds(r * 16, 16)] * wv)
            fire_scatter(t, b, bi)
            return 0

        lax.fori_loop(0, CH, chunk_body, 0)
        for t in range(CH - NB, CH):
            wait_scatter(t, t % NB, t % NBI)
        plsc.subcore_barrier()
        rows = pl.ds(sid * ROWS_PT, ROWS_PT)
        pltpu.sync_copy(acc.at[rows], out_hbm.at[cid, rows])

    return k(hp, row_p, col3_p, ew_p)


def _dis_of(dg_block):
    d = jnp.sum(dg_block, axis=0) + 1.0
    return jnp.where(d > 0, lax.rsqrt(d), 0.0)


def _mm_scale(xp, W, degp):
    def body(x_ref, w_ref, dg_ref, o_ref):
        dis = _dis_of(dg_ref[...])
        h = jnp.dot(x_ref[...], w_ref[...], preferred_element_type=F32)
        o_ref[...] = h * dis[:, None]

    return pl.pallas_call(
        body,
        grid=(NP // BR,),
        in_specs=[
            pl.BlockSpec((BR, D), lambda i: (i, 0)),
            pl.BlockSpec((D, D), lambda i: (0, 0)),
            pl.BlockSpec((NW, BR), lambda i: (0, i)),
        ],
        out_specs=pl.BlockSpec((BR, D), lambda i: (i, 0)),
        out_shape=jax.ShapeDtypeStruct((NP, D), F32),
    )(xp, W, degp)


def _agg_relu_mm_scale(parts, hp, degp, W):
    def body(p_ref, hp_ref, dg_ref, w_ref, o_ref):
        dis = _dis_of(dg_ref[...])
        s = jnp.sum(p_ref[...], axis=0) + hp_ref[...]
        z = jnp.maximum(s * dis[:, None], 0.0)
        h = jnp.dot(z, w_ref[...], preferred_element_type=F32)
        o_ref[...] = h * dis[:, None]

    return pl.pallas_call(
        body,
        grid=(NP // BR,),
        in_specs=[
            pl.BlockSpec((NC, BR, D), lambda i: (0, i, 0)),
            pl.BlockSpec((BR, D), lambda i: (i, 0)),
            pl.BlockSpec((NW, BR), lambda i: (0, i)),
            pl.BlockSpec((D, D), lambda i: (0, 0)),
        ],
        out_specs=pl.BlockSpec((BR, D), lambda i: (i, 0)),
        out_shape=jax.ShapeDtypeStruct((NP, D), F32),
    )(parts, hp, degp, W)


def _agg_log_softmax(parts, hp, degp):
    def body(p_ref, hp_ref, dg_ref, o_ref):
        dis = _dis_of(dg_ref[...])
        s = jnp.sum(p_ref[...], axis=0) + hp_ref[...]
        agg = s * dis[:, None]
        m = jnp.max(agg, axis=1, keepdims=True)
        sh = agg - m
        lse = jnp.log(jnp.sum(jnp.exp(sh), axis=1, keepdims=True))
        o_ref[...] = sh - lse

    return pl.pallas_call(
        body,
        grid=(NP // BR,),
        in_specs=[
            pl.BlockSpec((NC, BR, D), lambda i: (0, i, 0)),
            pl.BlockSpec((BR, D), lambda i: (i, 0)),
            pl.BlockSpec((NW, BR), lambda i: (0, i)),
        ],
        out_specs=pl.BlockSpec((BR, D), lambda i: (i, 0)),
        out_shape=jax.ShapeDtypeStruct((NP, D), F32),
    )(parts, hp, degp)


def kernel(x, edge_index, edge_weight, W1, W2):
    row = edge_index[0].astype(jnp.int32)
    col = edge_index[1].astype(jnp.int32)
    pad_e = EP - E
    row_p = jnp.concatenate([row, jnp.zeros((pad_e,), jnp.int32)])
    col_p = jnp.concatenate([col, jnp.zeros((pad_e,), jnp.int32)])
    ew_p = jnp.concatenate([edge_weight.astype(F32), jnp.zeros((pad_e,), F32)])
    xp = jnp.concatenate([x.astype(F32), jnp.zeros((NP - N, D), F32)], axis=0)
    col3_p = col_p.reshape(NW, CH, 1, K)

    degp = _deg_partials(col_p, ew_p)
    hp1 = _mm_scale(xp, W1, degp)
    parts1 = _aggregate(hp1, row_p, col3_p, ew_p)
    hp2 = _agg_relu_mm_scale(parts1, hp1, degp, W2)
    parts2 = _aggregate(hp2, row_p, col3_p, ew_p)
    outp = _agg_log_softmax(parts2, hp2, degp)
    return outp[:N]
